# baseline (speedup 1.0000x reference)
import jax
import jax.numpy as jnp
from jax.experimental import pallas as pl
from jax.experimental.pallas import tpu as pltpu

_B = 256
_D = 768
_P = 128
_K = 5
_TEMP = 1.6
_MAX_REF = 1000.0
import math as _math
_H_THRESH = _math.sin(_MAX_REF / (2.0 * 6371.0)) ** 2


def _dist_kernel(cand_ref, emb_ref, protos_ref, coords_ref,
                 minsq_ref, lng_ref, lat_ref):
    del cand_ref
    e = emb_ref[0]
    pr = protos_ref[0]
    diff = pr - e
    sums = jnp.sum(diff * diff, axis=1, keepdims=True)
    minv = jnp.min(sums)
    ii = jax.lax.broadcasted_iota(jnp.int32, (_P, 1), 0)
    amin = jnp.min(jnp.where(sums == minv, ii, _P))
    c = coords_ref[0]
    mask = ii == amin
    lng = jnp.sum(jnp.where(mask, c[:, 0:1], 0.0))
    lat = jnp.sum(jnp.where(mask, c[:, 1:2], 0.0))
    minsq_ref[...] = jnp.broadcast_to(minv, minsq_ref.shape)
    lng_ref[...] = jnp.broadcast_to(lng, lng_ref.shape)
    lat_ref[...] = jnp.broadcast_to(lat, lat_ref.shape)


def _epilogue_kernel(minsq_ref, lng_ref, lat_ref, cprobs_ref, ip_ref,
                     llh_ref, pid_ref, fprobs_ref):
    minsq = minsq_ref[...]
    lngs = lng_ref[...]
    lats = lat_ref[...]
    cprobs = cprobs_ref[...]
    ip = ip_ref[...]

    td = -jnp.sqrt(minsq + 1e-12)
    z = td / _TEMP
    zmax = jnp.max(z, axis=1, keepdims=True)
    ez = jnp.exp(z - zmax)
    probs = ez / jnp.sum(ez, axis=1, keepdims=True)
    fp = cprobs * probs

    jj = jax.lax.broadcasted_iota(jnp.int32, (_B, _K), 1)

    fmax = jnp.max(fp, axis=1, keepdims=True)
    rg = jnp.min(jnp.where(fp == fmax, jj, _K), axis=1, keepdims=True)
    sel = jj == rg
    r_lng = jnp.sum(jnp.where(sel, lngs, 0.0), axis=1, keepdims=True)
    r_lat = jnp.sum(jnp.where(sel, lats, 0.0), axis=1, keepdims=True)

    r = jnp.pi / 180.0
    lng1 = ip[:, 0:1] * r
    lat1 = ip[:, 1:2] * r
    lng2 = r_lng * r
    lat2 = r_lat * r
    h = (jnp.sin((lat2 - lat1) * 0.5) ** 2
         + jnp.cos(lat1) * jnp.cos(lat2) * jnp.sin((lng2 - lng1) * 0.5) ** 2)
    far = jnp.clip(h, 0.0, 1.0) > _H_THRESH

    fp2 = jnp.where(far, cprobs, fp)
    fmax2 = jnp.max(fp2, axis=1, keepdims=True)
    pid = jnp.min(jnp.where(fp2 == fmax2, jj, _K), axis=1, keepdims=True)
    sel2 = jj == pid
    f_lng = jnp.sum(jnp.where(sel2, lngs, 0.0), axis=1, keepdims=True)
    f_lat = jnp.sum(jnp.where(sel2, lats, 0.0), axis=1, keepdims=True)

    llh_ref[:, 0:1] = f_lng
    llh_ref[:, 1:2] = f_lat
    pid_ref[...] = pid
    fprobs_ref[...] = fp2


def kernel(embedding, initial_preds, candidate_cells, candidate_probs,
           protos, proto_coords):
    if embedding.ndim == 3:
        embedding = embedding.mean(axis=1)
    B, K = _B, _K
    cand = candidate_cells[:, :K].reshape(-1).astype(jnp.int32)
    emb3 = embedding.reshape(B, 1, _D)

    n = B * K
    grid_spec = pltpu.PrefetchScalarGridSpec(
        num_scalar_prefetch=1,
        grid=(n,),
        in_specs=[
            pl.BlockSpec((1, 1, _D), lambda i, c: (i // _K, 0, 0)),
            pl.BlockSpec((1, _P, _D), lambda i, c: (c[i], 0, 0)),
            pl.BlockSpec((1, _P, 2), lambda i, c: (c[i], 0, 0)),
        ],
        out_specs=[
            pl.BlockSpec((1, 1, 128), lambda i, c: (i, 0, 0)),
            pl.BlockSpec((1, 1, 128), lambda i, c: (i, 0, 0)),
            pl.BlockSpec((1, 1, 128), lambda i, c: (i, 0, 0)),
        ],
    )
    minsq, lng, lat = pl.pallas_call(
        _dist_kernel,
        grid_spec=grid_spec,
        out_shape=[
            jax.ShapeDtypeStruct((n, 1, 128), jnp.float32),
            jax.ShapeDtypeStruct((n, 1, 128), jnp.float32),
            jax.ShapeDtypeStruct((n, 1, 128), jnp.float32),
        ],
    )(cand, emb3, protos, proto_coords)

    minsq_bk = minsq[:, 0, 0].reshape(B, K)
    lngs_bk = lng[:, 0, 0].reshape(B, K)
    lats_bk = lat[:, 0, 0].reshape(B, K)

    llh, pid, fprobs = pl.pallas_call(
        _epilogue_kernel,
        out_shape=[
            jax.ShapeDtypeStruct((B, 2), jnp.float32),
            jax.ShapeDtypeStruct((B, 1), jnp.int32),
            jax.ShapeDtypeStruct((B, K), jnp.float32),
        ],
    )(minsq_bk, lngs_bk, lats_bk, candidate_probs[:, :K].astype(jnp.float32),
      initial_preds)

    return llh, pid[:, 0], fprobs


# speedup vs baseline: 1.2294x; 1.2294x over previous
import math as _math

import jax
import jax.numpy as jnp
from jax.experimental import pallas as pl
from jax.experimental.pallas import tpu as pltpu

_B = 256
_D = 768
_G = 1000
_P = 128
_K = 5
_TEMP = 1.6
_MAX_REF = 1000.0
_H_THRESH = _math.sin(_MAX_REF / (2.0 * 6371.0)) ** 2


def _dist_kernel(sc_ref, ob_ref, emb_ref, protos_ref, coords_ref, out_ref):
    i = pl.program_id(0)
    r = ob_ref[i]
    b = r // _K
    g = sc_ref[i]
    e = emb_ref[pl.ds(b, 1), :]
    pr = protos_ref[0]
    diff = pr - e
    sums = jnp.sum(diff * diff, axis=1, keepdims=True)
    minv = jnp.min(sums)
    ii = jax.lax.broadcasted_iota(jnp.int32, (_P, 1), 0)
    amin = jnp.min(jnp.where(sums == minv, ii, _P))
    lane = jax.lax.broadcasted_iota(jnp.int32, (1, 128), 1)
    lmask = lane == amin
    lngrow = coords_ref[pl.ds(2 * g, 1), :]
    latrow = coords_ref[pl.ds(2 * g + 1, 1), :]
    lng = jnp.sum(jnp.where(lmask, lngrow, 0.0))
    lat = jnp.sum(jnp.where(lmask, latrow, 0.0))
    row = jnp.where(lane == 0, minv,
                    jnp.where(lane == 1, lng,
                              jnp.where(lane == 2, lat, 0.0)))
    out_ref[pl.ds(r, 1), :] = row


def _epilogue_kernel(minsq_ref, lng_ref, lat_ref, cprobs_ref, ip_ref,
                     llh_ref, pid_ref, fprobs_ref):
    minsq = minsq_ref[...]
    lngs = lng_ref[...]
    lats = lat_ref[...]
    cprobs = cprobs_ref[...]
    ip = ip_ref[...]

    td = -jnp.sqrt(minsq + 1e-12)
    z = td / _TEMP
    zmax = jnp.max(z, axis=1, keepdims=True)
    ez = jnp.exp(z - zmax)
    probs = ez / jnp.sum(ez, axis=1, keepdims=True)
    fp = cprobs * probs

    jj = jax.lax.broadcasted_iota(jnp.int32, (_B, _K), 1)

    fmax = jnp.max(fp, axis=1, keepdims=True)
    rg = jnp.min(jnp.where(fp == fmax, jj, _K), axis=1, keepdims=True)
    sel = jj == rg
    r_lng = jnp.sum(jnp.where(sel, lngs, 0.0), axis=1, keepdims=True)
    r_lat = jnp.sum(jnp.where(sel, lats, 0.0), axis=1, keepdims=True)

    r = jnp.pi / 180.0
    lng1 = ip[:, 0:1] * r
    lat1 = ip[:, 1:2] * r
    lng2 = r_lng * r
    lat2 = r_lat * r
    h = (jnp.sin((lat2 - lat1) * 0.5) ** 2
         + jnp.cos(lat1) * jnp.cos(lat2) * jnp.sin((lng2 - lng1) * 0.5) ** 2)
    far = jnp.clip(h, 0.0, 1.0) > _H_THRESH

    fp2 = jnp.where(far, cprobs, fp)
    fmax2 = jnp.max(fp2, axis=1, keepdims=True)
    pid = jnp.min(jnp.where(fp2 == fmax2, jj, _K), axis=1, keepdims=True)
    sel2 = jj == pid
    f_lng = jnp.sum(jnp.where(sel2, lngs, 0.0), axis=1, keepdims=True)
    f_lat = jnp.sum(jnp.where(sel2, lats, 0.0), axis=1, keepdims=True)

    llh_ref[:, 0:1] = f_lng
    llh_ref[:, 1:2] = f_lat
    pid_ref[...] = pid
    fprobs_ref[...] = fp2


def kernel(embedding, initial_preds, candidate_cells, candidate_probs,
           protos, proto_coords):
    if embedding.ndim == 3:
        embedding = embedding.mean(axis=1)
    B, K = _B, _K
    n = B * K
    cand = candidate_cells[:, :K].reshape(-1).astype(jnp.int32)
    order = jnp.argsort(cand).astype(jnp.int32)
    sc = jnp.take(cand, order)
    coords_t = proto_coords.transpose(0, 2, 1).reshape(2 * _G, _P)

    grid_spec = pltpu.PrefetchScalarGridSpec(
        num_scalar_prefetch=2,
        grid=(n,),
        in_specs=[
            pl.BlockSpec((B, _D), lambda i, s, o: (0, 0)),
            pl.BlockSpec((1, _P, _D), lambda i, s, o: (s[i], 0, 0)),
            pl.BlockSpec((2 * _G, _P), lambda i, s, o: (0, 0)),
        ],
        out_specs=pl.BlockSpec((n, 128), lambda i, s, o: (0, 0)),
    )
    out = pl.pallas_call(
        _dist_kernel,
        grid_spec=grid_spec,
        out_shape=jax.ShapeDtypeStruct((n, 128), jnp.float32),
    )(sc, order, embedding, protos, coords_t)

    minsq_bk = out[:, 0].reshape(B, K)
    lngs_bk = out[:, 1].reshape(B, K)
    lats_bk = out[:, 2].reshape(B, K)

    llh, pid, fprobs = pl.pallas_call(
        _epilogue_kernel,
        out_shape=[
            jax.ShapeDtypeStruct((B, 2), jnp.float32),
            jax.ShapeDtypeStruct((B, 1), jnp.int32),
            jax.ShapeDtypeStruct((B, K), jnp.float32),
        ],
    )(minsq_bk, lngs_bk, lats_bk, candidate_probs[:, :K].astype(jnp.float32),
      initial_preds)

    return llh, pid[:, 0], fprobs


# speedup vs baseline: 2.6932x; 2.1907x over previous
import math as _math

import jax
import jax.numpy as jnp
from jax.experimental import pallas as pl
from jax.experimental.pallas import tpu as pltpu

_B = 256
_D = 768
_G = 1000
_P = 128
_K = 5
_TEMP = 1.6
_MAX_REF = 1000.0
_H_THRESH = _math.sin(_MAX_REF / (2.0 * 6371.0)) ** 2


_W = 4
_CHUNK = (_B * _K) // _W


def _dist_kernel(sc_ref, ob_ref, emb_ref, p0_ref, p1_ref, p2_ref, p3_ref,
                 coords_ref, out_ref):
    i = pl.program_id(0)
    for j, p_ref in enumerate((p0_ref, p1_ref, p2_ref, p3_ref)):
        idx = j * _CHUNK + i
        r = ob_ref[idx]
        b = r // _K
        g = sc_ref[idx]
        e = emb_ref[pl.ds(b, 1), :]
        pr = p_ref[0]
        diff = pr - e
        sums = jnp.sum(diff * diff, axis=1, keepdims=True)
        minv = jnp.min(sums)
        ii = jax.lax.broadcasted_iota(jnp.int32, (_P, 1), 0)
        amin = jnp.min(jnp.where(sums == minv, ii, _P))
        lane = jax.lax.broadcasted_iota(jnp.int32, (1, 128), 1)
        lmask = lane == amin
        lngrow = coords_ref[pl.ds(2 * g, 1), :]
        latrow = coords_ref[pl.ds(2 * g + 1, 1), :]
        lng = jnp.sum(jnp.where(lmask, lngrow, 0.0))
        lat = jnp.sum(jnp.where(lmask, latrow, 0.0))
        row = jnp.where(lane == 0, minv,
                        jnp.where(lane == 1, lng,
                                  jnp.where(lane == 2, lat, 0.0)))
        out_ref[pl.ds(r, 1), :] = row


def _epilogue_kernel(minsq_ref, lng_ref, lat_ref, cprobs_ref, ip_ref,
                     llh_ref, pid_ref, fprobs_ref):
    minsq = minsq_ref[...]
    lngs = lng_ref[...]
    lats = lat_ref[...]
    cprobs = cprobs_ref[...]
    ip = ip_ref[...]

    td = -jnp.sqrt(minsq + 1e-12)
    z = td / _TEMP
    zmax = jnp.max(z, axis=1, keepdims=True)
    ez = jnp.exp(z - zmax)
    probs = ez / jnp.sum(ez, axis=1, keepdims=True)
    fp = cprobs * probs

    jj = jax.lax.broadcasted_iota(jnp.int32, (_B, _K), 1)

    fmax = jnp.max(fp, axis=1, keepdims=True)
    rg = jnp.min(jnp.where(fp == fmax, jj, _K), axis=1, keepdims=True)
    sel = jj == rg
    r_lng = jnp.sum(jnp.where(sel, lngs, 0.0), axis=1, keepdims=True)
    r_lat = jnp.sum(jnp.where(sel, lats, 0.0), axis=1, keepdims=True)

    r = jnp.pi / 180.0
    lng1 = ip[:, 0:1] * r
    lat1 = ip[:, 1:2] * r
    lng2 = r_lng * r
    lat2 = r_lat * r
    h = (jnp.sin((lat2 - lat1) * 0.5) ** 2
         + jnp.cos(lat1) * jnp.cos(lat2) * jnp.sin((lng2 - lng1) * 0.5) ** 2)
    far = jnp.clip(h, 0.0, 1.0) > _H_THRESH

    fp2 = jnp.where(far, cprobs, fp)
    fmax2 = jnp.max(fp2, axis=1, keepdims=True)
    pid = jnp.min(jnp.where(fp2 == fmax2, jj, _K), axis=1, keepdims=True)
    sel2 = jj == pid
    f_lng = jnp.sum(jnp.where(sel2, lngs, 0.0), axis=1, keepdims=True)
    f_lat = jnp.sum(jnp.where(sel2, lats, 0.0), axis=1, keepdims=True)

    llh_ref[:, 0:1] = f_lng
    llh_ref[:, 1:2] = f_lat
    pid_ref[...] = pid
    fprobs_ref[...] = fp2


def kernel(embedding, initial_preds, candidate_cells, candidate_probs,
           protos, proto_coords):
    if embedding.ndim == 3:
        embedding = embedding.mean(axis=1)
    B, K = _B, _K
    n = B * K
    cand = candidate_cells[:, :K].reshape(-1).astype(jnp.int32)
    order = jnp.argsort(cand).astype(jnp.int32)
    sc = jnp.take(cand, order)
    coords_t = proto_coords.transpose(0, 2, 1).reshape(2 * _G, _P)

    grid_spec = pltpu.PrefetchScalarGridSpec(
        num_scalar_prefetch=2,
        grid=(_CHUNK,),
        in_specs=[
            pl.BlockSpec((B, _D), lambda i, s, o: (0, 0)),
            pl.BlockSpec((1, _P, _D), lambda i, s, o: (s[i], 0, 0)),
            pl.BlockSpec((1, _P, _D), lambda i, s, o: (s[_CHUNK + i], 0, 0)),
            pl.BlockSpec((1, _P, _D),
                         lambda i, s, o: (s[2 * _CHUNK + i], 0, 0)),
            pl.BlockSpec((1, _P, _D),
                         lambda i, s, o: (s[3 * _CHUNK + i], 0, 0)),
            pl.BlockSpec((2 * _G, _P), lambda i, s, o: (0, 0)),
        ],
        out_specs=pl.BlockSpec((n, 128), lambda i, s, o: (0, 0)),
    )
    out = pl.pallas_call(
        _dist_kernel,
        grid_spec=grid_spec,
        out_shape=jax.ShapeDtypeStruct((n, 128), jnp.float32),
    )(sc, order, embedding, protos, protos, protos, protos, coords_t)

    minsq_bk = out[:, 0].reshape(B, K)
    lngs_bk = out[:, 1].reshape(B, K)
    lats_bk = out[:, 2].reshape(B, K)

    llh, pid, fprobs = pl.pallas_call(
        _epilogue_kernel,
        out_shape=[
            jax.ShapeDtypeStruct((B, 2), jnp.float32),
            jax.ShapeDtypeStruct((B, 1), jnp.int32),
            jax.ShapeDtypeStruct((B, K), jnp.float32),
        ],
    )(minsq_bk, lngs_bk, lats_bk, candidate_probs[:, :K].astype(jnp.float32),
      initial_preds)

    return llh, pid[:, 0], fprobs


# speedup vs baseline: 3.2956x; 1.2237x over previous
import math as _math

import jax
import jax.numpy as jnp
from jax.experimental import pallas as pl
from jax.experimental.pallas import tpu as pltpu

_B = 256
_D = 768
_G = 1000
_P = 128
_K = 5
_TEMP = 1.6
_MAX_REF = 1000.0
_H_THRESH = _math.sin(_MAX_REF / (2.0 * 6371.0)) ** 2


_W = 8
_CHUNK = (_B * _K) // _W


def _dist_kernel(sc_ref, ob_ref, emb_ref, *rest):
    p_refs = rest[:_W]
    coords_ref = rest[_W]
    out_ref = rest[_W + 1]
    i = pl.program_id(0)
    for j, p_ref in enumerate(p_refs):
        idx = j * _CHUNK + i
        r = ob_ref[idx]
        b = r // _K
        g = sc_ref[idx]
        e = emb_ref[pl.ds(b, 1), :]
        pr = p_ref[0]
        diff = pr - e
        sums = jnp.sum(diff * diff, axis=1, keepdims=True)
        minv = jnp.min(sums)
        ii = jax.lax.broadcasted_iota(jnp.int32, (_P, 1), 0)
        amin = jnp.min(jnp.where(sums == minv, ii, _P))
        lane = jax.lax.broadcasted_iota(jnp.int32, (1, 128), 1)
        lmask = lane == amin
        lngrow = coords_ref[pl.ds(2 * g, 1), :]
        latrow = coords_ref[pl.ds(2 * g + 1, 1), :]
        lng = jnp.sum(jnp.where(lmask, lngrow, 0.0))
        lat = jnp.sum(jnp.where(lmask, latrow, 0.0))
        row = jnp.where(lane == 0, minv,
                        jnp.where(lane == 1, lng,
                                  jnp.where(lane == 2, lat, 0.0)))
        out_ref[pl.ds(r, 1), :] = row


def _epilogue_kernel(minsq_ref, lng_ref, lat_ref, cprobs_ref, ip_ref,
                     llh_ref, pid_ref, fprobs_ref):
    minsq = minsq_ref[...]
    lngs = lng_ref[...]
    lats = lat_ref[...]
    cprobs = cprobs_ref[...]
    ip = ip_ref[...]

    td = -jnp.sqrt(minsq + 1e-12)
    z = td / _TEMP
    zmax = jnp.max(z, axis=1, keepdims=True)
    ez = jnp.exp(z - zmax)
    probs = ez / jnp.sum(ez, axis=1, keepdims=True)
    fp = cprobs * probs

    jj = jax.lax.broadcasted_iota(jnp.int32, (_B, _K), 1)

    fmax = jnp.max(fp, axis=1, keepdims=True)
    rg = jnp.min(jnp.where(fp == fmax, jj, _K), axis=1, keepdims=True)
    sel = jj == rg
    r_lng = jnp.sum(jnp.where(sel, lngs, 0.0), axis=1, keepdims=True)
    r_lat = jnp.sum(jnp.where(sel, lats, 0.0), axis=1, keepdims=True)

    r = jnp.pi / 180.0
    lng1 = ip[:, 0:1] * r
    lat1 = ip[:, 1:2] * r
    lng2 = r_lng * r
    lat2 = r_lat * r
    h = (jnp.sin((lat2 - lat1) * 0.5) ** 2
         + jnp.cos(lat1) * jnp.cos(lat2) * jnp.sin((lng2 - lng1) * 0.5) ** 2)
    far = jnp.clip(h, 0.0, 1.0) > _H_THRESH

    fp2 = jnp.where(far, cprobs, fp)
    fmax2 = jnp.max(fp2, axis=1, keepdims=True)
    pid = jnp.min(jnp.where(fp2 == fmax2, jj, _K), axis=1, keepdims=True)
    sel2 = jj == pid
    f_lng = jnp.sum(jnp.where(sel2, lngs, 0.0), axis=1, keepdims=True)
    f_lat = jnp.sum(jnp.where(sel2, lats, 0.0), axis=1, keepdims=True)

    llh_ref[:, 0:1] = f_lng
    llh_ref[:, 1:2] = f_lat
    pid_ref[...] = pid
    fprobs_ref[...] = fp2


def kernel(embedding, initial_preds, candidate_cells, candidate_probs,
           protos, proto_coords):
    if embedding.ndim == 3:
        embedding = embedding.mean(axis=1)
    B, K = _B, _K
    n = B * K
    cand = candidate_cells[:, :K].reshape(-1).astype(jnp.int32)
    order = jnp.argsort(cand).astype(jnp.int32)
    sc = jnp.take(cand, order)
    coords_t = proto_coords.transpose(0, 2, 1).reshape(2 * _G, _P)

    grid_spec = pltpu.PrefetchScalarGridSpec(
        num_scalar_prefetch=2,
        grid=(_CHUNK,),
        in_specs=(
            [pl.BlockSpec((B, _D), lambda i, s, o: (0, 0))]
            + [pl.BlockSpec((1, _P, _D),
                            lambda i, s, o, j=j: (s[j * _CHUNK + i], 0, 0))
               for j in range(_W)]
            + [pl.BlockSpec((2 * _G, _P), lambda i, s, o: (0, 0))]
        ),
        out_specs=pl.BlockSpec((n, 128), lambda i, s, o: (0, 0)),
    )
    out = pl.pallas_call(
        _dist_kernel,
        grid_spec=grid_spec,
        out_shape=jax.ShapeDtypeStruct((n, 128), jnp.float32),
    )(sc, order, embedding, *([protos] * _W), coords_t)

    minsq_bk = out[:, 0].reshape(B, K)
    lngs_bk = out[:, 1].reshape(B, K)
    lats_bk = out[:, 2].reshape(B, K)

    llh, pid, fprobs = pl.pallas_call(
        _epilogue_kernel,
        out_shape=[
            jax.ShapeDtypeStruct((B, 2), jnp.float32),
            jax.ShapeDtypeStruct((B, 1), jnp.int32),
            jax.ShapeDtypeStruct((B, K), jnp.float32),
        ],
    )(minsq_bk, lngs_bk, lats_bk, candidate_probs[:, :K].astype(jnp.float32),
      initial_preds)

    return llh, pid[:, 0], fprobs


# speedup vs baseline: 3.4467x; 1.0458x over previous
import math as _math

import jax
import jax.numpy as jnp
from jax.experimental import pallas as pl
from jax.experimental.pallas import tpu as pltpu

_B = 256
_D = 768
_G = 1000
_P = 128
_K = 5
_TEMP = 1.6
_MAX_REF = 1000.0
_H_THRESH = _math.sin(_MAX_REF / (2.0 * 6371.0)) ** 2


_W = 16
_CHUNK = (_B * _K) // _W


def _dist_kernel(sc_ref, ob_ref, emb_ref, *rest):
    p_refs = rest[:_W]
    coords_ref = rest[_W]
    out_ref = rest[_W + 1]
    i = pl.program_id(0)
    for j, p_ref in enumerate(p_refs):
        idx = j * _CHUNK + i
        r = ob_ref[idx]
        b = r // _K
        g = sc_ref[idx]
        e = emb_ref[pl.ds(b, 1), :]
        pr = p_ref[0]
        diff = pr - e
        sums = jnp.sum(diff * diff, axis=1, keepdims=True)
        minv = jnp.min(sums)
        ii = jax.lax.broadcasted_iota(jnp.int32, (_P, 1), 0)
        amin = jnp.min(jnp.where(sums == minv, ii, _P))
        lane = jax.lax.broadcasted_iota(jnp.int32, (1, 128), 1)
        lmask = lane == amin
        lngrow = coords_ref[pl.ds(2 * g, 1), :]
        latrow = coords_ref[pl.ds(2 * g + 1, 1), :]
        lng = jnp.sum(jnp.where(lmask, lngrow, 0.0))
        lat = jnp.sum(jnp.where(lmask, latrow, 0.0))
        row = jnp.where(lane == 0, minv,
                        jnp.where(lane == 1, lng,
                                  jnp.where(lane == 2, lat, 0.0)))
        out_ref[pl.ds(r, 1), :] = row


def _epilogue_kernel(minsq_ref, lng_ref, lat_ref, cprobs_ref, ip_ref,
                     llh_ref, pid_ref, fprobs_ref):
    minsq = minsq_ref[...]
    lngs = lng_ref[...]
    lats = lat_ref[...]
    cprobs = cprobs_ref[...]
    ip = ip_ref[...]

    td = -jnp.sqrt(minsq + 1e-12)
    z = td / _TEMP
    zmax = jnp.max(z, axis=1, keepdims=True)
    ez = jnp.exp(z - zmax)
    probs = ez / jnp.sum(ez, axis=1, keepdims=True)
    fp = cprobs * probs

    jj = jax.lax.broadcasted_iota(jnp.int32, (_B, _K), 1)

    fmax = jnp.max(fp, axis=1, keepdims=True)
    rg = jnp.min(jnp.where(fp == fmax, jj, _K), axis=1, keepdims=True)
    sel = jj == rg
    r_lng = jnp.sum(jnp.where(sel, lngs, 0.0), axis=1, keepdims=True)
    r_lat = jnp.sum(jnp.where(sel, lats, 0.0), axis=1, keepdims=True)

    r = jnp.pi / 180.0
    lng1 = ip[:, 0:1] * r
    lat1 = ip[:, 1:2] * r
    lng2 = r_lng * r
    lat2 = r_lat * r
    h = (jnp.sin((lat2 - lat1) * 0.5) ** 2
         + jnp.cos(lat1) * jnp.cos(lat2) * jnp.sin((lng2 - lng1) * 0.5) ** 2)
    far = jnp.clip(h, 0.0, 1.0) > _H_THRESH

    fp2 = jnp.where(far, cprobs, fp)
    fmax2 = jnp.max(fp2, axis=1, keepdims=True)
    pid = jnp.min(jnp.where(fp2 == fmax2, jj, _K), axis=1, keepdims=True)
    sel2 = jj == pid
    f_lng = jnp.sum(jnp.where(sel2, lngs, 0.0), axis=1, keepdims=True)
    f_lat = jnp.sum(jnp.where(sel2, lats, 0.0), axis=1, keepdims=True)

    llh_ref[:, 0:1] = f_lng
    llh_ref[:, 1:2] = f_lat
    pid_ref[...] = pid
    fprobs_ref[...] = fp2


def kernel(embedding, initial_preds, candidate_cells, candidate_probs,
           protos, proto_coords):
    if embedding.ndim == 3:
        embedding = embedding.mean(axis=1)
    B, K = _B, _K
    n = B * K
    cand = candidate_cells[:, :K].reshape(-1).astype(jnp.int32)
    order = jnp.argsort(cand).astype(jnp.int32)
    sc = jnp.take(cand, order)
    coords_t = proto_coords.transpose(0, 2, 1).reshape(2 * _G, _P)

    grid_spec = pltpu.PrefetchScalarGridSpec(
        num_scalar_prefetch=2,
        grid=(_CHUNK,),
        in_specs=(
            [pl.BlockSpec((B, _D), lambda i, s, o: (0, 0))]
            + [pl.BlockSpec((1, _P, _D),
                            lambda i, s, o, j=j: (s[j * _CHUNK + i], 0, 0))
               for j in range(_W)]
            + [pl.BlockSpec((2 * _G, _P), lambda i, s, o: (0, 0))]
        ),
        out_specs=pl.BlockSpec((n, 128), lambda i, s, o: (0, 0)),
    )
    out = pl.pallas_call(
        _dist_kernel,
        grid_spec=grid_spec,
        out_shape=jax.ShapeDtypeStruct((n, 128), jnp.float32),
    )(sc, order, embedding, *([protos] * _W), coords_t)

    minsq_bk = out[:, 0].reshape(B, K)
    lngs_bk = out[:, 1].reshape(B, K)
    lats_bk = out[:, 2].reshape(B, K)

    llh, pid, fprobs = pl.pallas_call(
        _epilogue_kernel,
        out_shape=[
            jax.ShapeDtypeStruct((B, 2), jnp.float32),
            jax.ShapeDtypeStruct((B, 1), jnp.int32),
            jax.ShapeDtypeStruct((B, K), jnp.float32),
        ],
    )(minsq_bk, lngs_bk, lats_bk, candidate_probs[:, :K].astype(jnp.float32),
      initial_preds)

    return llh, pid[:, 0], fprobs


# speedup vs baseline: 3.4902x; 1.0126x over previous
import math as _math

import jax
import jax.numpy as jnp
from jax.experimental import pallas as pl
from jax.experimental.pallas import tpu as pltpu

_B = 256
_D = 768
_G = 1000
_P = 128
_K = 5
_TEMP = 1.6
_MAX_REF = 1000.0
_H_THRESH = _math.sin(_MAX_REF / (2.0 * 6371.0)) ** 2


_W = 32
_CHUNK = (_B * _K) // _W


def _dist_kernel(sc_ref, ob_ref, emb_ref, *rest):
    p_refs = rest[:_W]
    coords_ref = rest[_W]
    out_ref = rest[_W + 1]
    i = pl.program_id(0)
    for j, p_ref in enumerate(p_refs):
        idx = j * _CHUNK + i
        r = ob_ref[idx]
        b = r // _K
        g = sc_ref[idx]
        e = emb_ref[pl.ds(b, 1), :]
        pr = p_ref[0]
        diff = pr - e
        sums = jnp.sum(diff * diff, axis=1, keepdims=True)
        minv = jnp.min(sums)
        ii = jax.lax.broadcasted_iota(jnp.int32, (_P, 1), 0)
        amin = jnp.min(jnp.where(sums == minv, ii, _P))
        lane = jax.lax.broadcasted_iota(jnp.int32, (1, 128), 1)
        lmask = lane == amin
        lngrow = coords_ref[pl.ds(2 * g, 1), :]
        latrow = coords_ref[pl.ds(2 * g + 1, 1), :]
        lng = jnp.sum(jnp.where(lmask, lngrow, 0.0))
        lat = jnp.sum(jnp.where(lmask, latrow, 0.0))
        row = jnp.where(lane == 0, minv,
                        jnp.where(lane == 1, lng,
                                  jnp.where(lane == 2, lat, 0.0)))
        out_ref[pl.ds(r, 1), :] = row


def _epilogue_kernel(minsq_ref, lng_ref, lat_ref, cprobs_ref, ip_ref,
                     llh_ref, pid_ref, fprobs_ref):
    minsq = minsq_ref[...]
    lngs = lng_ref[...]
    lats = lat_ref[...]
    cprobs = cprobs_ref[...]
    ip = ip_ref[...]

    td = -jnp.sqrt(minsq + 1e-12)
    z = td / _TEMP
    zmax = jnp.max(z, axis=1, keepdims=True)
    ez = jnp.exp(z - zmax)
    probs = ez / jnp.sum(ez, axis=1, keepdims=True)
    fp = cprobs * probs

    jj = jax.lax.broadcasted_iota(jnp.int32, (_B, _K), 1)

    fmax = jnp.max(fp, axis=1, keepdims=True)
    rg = jnp.min(jnp.where(fp == fmax, jj, _K), axis=1, keepdims=True)
    sel = jj == rg
    r_lng = jnp.sum(jnp.where(sel, lngs, 0.0), axis=1, keepdims=True)
    r_lat = jnp.sum(jnp.where(sel, lats, 0.0), axis=1, keepdims=True)

    r = jnp.pi / 180.0
    lng1 = ip[:, 0:1] * r
    lat1 = ip[:, 1:2] * r
    lng2 = r_lng * r
    lat2 = r_lat * r
    h = (jnp.sin((lat2 - lat1) * 0.5) ** 2
         + jnp.cos(lat1) * jnp.cos(lat2) * jnp.sin((lng2 - lng1) * 0.5) ** 2)
    far = jnp.clip(h, 0.0, 1.0) > _H_THRESH

    fp2 = jnp.where(far, cprobs, fp)
    fmax2 = jnp.max(fp2, axis=1, keepdims=True)
    pid = jnp.min(jnp.where(fp2 == fmax2, jj, _K), axis=1, keepdims=True)
    sel2 = jj == pid
    f_lng = jnp.sum(jnp.where(sel2, lngs, 0.0), axis=1, keepdims=True)
    f_lat = jnp.sum(jnp.where(sel2, lats, 0.0), axis=1, keepdims=True)

    llh_ref[:, 0:1] = f_lng
    llh_ref[:, 1:2] = f_lat
    pid_ref[...] = pid
    fprobs_ref[...] = fp2


def kernel(embedding, initial_preds, candidate_cells, candidate_probs,
           protos, proto_coords):
    if embedding.ndim == 3:
        embedding = embedding.mean(axis=1)
    B, K = _B, _K
    n = B * K
    cand = candidate_cells[:, :K].reshape(-1).astype(jnp.int32)
    order = jnp.argsort(cand).astype(jnp.int32)
    sc = jnp.take(cand, order)
    coords_t = proto_coords.transpose(0, 2, 1).reshape(2 * _G, _P)

    grid_spec = pltpu.PrefetchScalarGridSpec(
        num_scalar_prefetch=2,
        grid=(_CHUNK,),
        in_specs=(
            [pl.BlockSpec((B, _D), lambda i, s, o: (0, 0))]
            + [pl.BlockSpec((1, _P, _D),
                            lambda i, s, o, j=j: (s[j * _CHUNK + i], 0, 0))
               for j in range(_W)]
            + [pl.BlockSpec((2 * _G, _P), lambda i, s, o: (0, 0))]
        ),
        out_specs=pl.BlockSpec((n, 128), lambda i, s, o: (0, 0)),
    )
    out = pl.pallas_call(
        _dist_kernel,
        grid_spec=grid_spec,
        out_shape=jax.ShapeDtypeStruct((n, 128), jnp.float32),
    )(sc, order, embedding, *([protos] * _W), coords_t)

    minsq_bk = out[:, 0].reshape(B, K)
    lngs_bk = out[:, 1].reshape(B, K)
    lats_bk = out[:, 2].reshape(B, K)

    llh, pid, fprobs = pl.pallas_call(
        _epilogue_kernel,
        out_shape=[
            jax.ShapeDtypeStruct((B, 2), jnp.float32),
            jax.ShapeDtypeStruct((B, 1), jnp.int32),
            jax.ShapeDtypeStruct((B, K), jnp.float32),
        ],
    )(minsq_bk, lngs_bk, lats_bk, candidate_probs[:, :K].astype(jnp.float32),
      initial_preds)

    return llh, pid[:, 0], fprobs


# speedup vs baseline: 5.9161x; 1.6951x over previous
import math as _math

import jax
import jax.numpy as jnp
from jax.experimental import pallas as pl
from jax.experimental.pallas import tpu as pltpu

_B = 256
_D = 768
_G = 1000
_P = 128
_K = 5
_TEMP = 1.6
_MAX_REF = 1000.0
_H_THRESH = _math.sin(_MAX_REF / (2.0 * 6371.0)) ** 2


_W = 32
_CHUNK = (_B * _K) // _W


def _dist_kernel(sc_ref, ob_ref, emb_ref, *rest):
    p_refs = rest[:_W]
    coords_ref = rest[_W]
    out_ref = rest[_W + 1]
    i = pl.program_id(0)
    ii = jax.lax.broadcasted_iota(jnp.int32, (_P, 1), 0)
    lane = jax.lax.broadcasted_iota(jnp.int32, (1, 128), 1)

    def heavy(j):
        idx = j * _CHUNK + i
        r = ob_ref[idx]
        b = r // _K
        g = sc_ref[idx]
        e = emb_ref[pl.ds(b, 1), :]
        pr = p_refs[j][0]
        diff = pr - e
        sums = jnp.sum(diff * diff, axis=1, keepdims=True)
        crows = coords_ref[pl.ds(2 * g, 2), :]
        return r, sums, crows

    def tail(state):
        r, sums, crows = state
        minv = jnp.min(sums)
        amin = jnp.min(jnp.where(sums == minv, ii, _P))
        lmask = lane == amin
        ll = jnp.sum(jnp.where(lmask, crows, 0.0), axis=1, keepdims=True)
        lng = ll[0:1, :]
        lat = ll[1:2, :]
        row = jnp.where(lane == 0, minv,
                        jnp.where(lane == 1, lng,
                                  jnp.where(lane == 2, lat, 0.0)))
        out_ref[pl.ds(r, 1), :] = row

    prev = None
    for j in range(_W):
        cur = heavy(j)
        if prev is not None:
            tail(prev)
        prev = cur
    tail(prev)


def _epilogue_kernel(minsq_ref, lng_ref, lat_ref, cprobs_ref, ip_ref,
                     llh_ref, pid_ref, fprobs_ref):
    minsq = minsq_ref[...]
    lngs = lng_ref[...]
    lats = lat_ref[...]
    cprobs = cprobs_ref[...]
    ip = ip_ref[...]

    td = -jnp.sqrt(minsq + 1e-12)
    z = td / _TEMP
    zmax = jnp.max(z, axis=1, keepdims=True)
    ez = jnp.exp(z - zmax)
    probs = ez / jnp.sum(ez, axis=1, keepdims=True)
    fp = cprobs * probs

    jj = jax.lax.broadcasted_iota(jnp.int32, (_B, _K), 1)

    fmax = jnp.max(fp, axis=1, keepdims=True)
    rg = jnp.min(jnp.where(fp == fmax, jj, _K), axis=1, keepdims=True)
    sel = jj == rg
    r_lng = jnp.sum(jnp.where(sel, lngs, 0.0), axis=1, keepdims=True)
    r_lat = jnp.sum(jnp.where(sel, lats, 0.0), axis=1, keepdims=True)

    r = jnp.pi / 180.0
    lng1 = ip[:, 0:1] * r
    lat1 = ip[:, 1:2] * r
    lng2 = r_lng * r
    lat2 = r_lat * r
    h = (jnp.sin((lat2 - lat1) * 0.5) ** 2
         + jnp.cos(lat1) * jnp.cos(lat2) * jnp.sin((lng2 - lng1) * 0.5) ** 2)
    far = jnp.clip(h, 0.0, 1.0) > _H_THRESH

    fp2 = jnp.where(far, cprobs, fp)
    fmax2 = jnp.max(fp2, axis=1, keepdims=True)
    pid = jnp.min(jnp.where(fp2 == fmax2, jj, _K), axis=1, keepdims=True)
    sel2 = jj == pid
    f_lng = jnp.sum(jnp.where(sel2, lngs, 0.0), axis=1, keepdims=True)
    f_lat = jnp.sum(jnp.where(sel2, lats, 0.0), axis=1, keepdims=True)

    llh_ref[:, 0:1] = f_lng
    llh_ref[:, 1:2] = f_lat
    pid_ref[...] = pid
    fprobs_ref[...] = fp2


def kernel(embedding, initial_preds, candidate_cells, candidate_probs,
           protos, proto_coords):
    if embedding.ndim == 3:
        embedding = embedding.mean(axis=1)
    B, K = _B, _K
    n = B * K
    cand = candidate_cells[:, :K].reshape(-1).astype(jnp.int32)
    order = jnp.argsort(cand).astype(jnp.int32)
    sc = jnp.take(cand, order)
    coords_t = proto_coords.transpose(0, 2, 1).reshape(2 * _G, _P)

    grid_spec = pltpu.PrefetchScalarGridSpec(
        num_scalar_prefetch=2,
        grid=(_CHUNK,),
        in_specs=(
            [pl.BlockSpec((B, _D), lambda i, s, o: (0, 0))]
            + [pl.BlockSpec((1, _P, _D),
                            lambda i, s, o, j=j: (s[j * _CHUNK + i], 0, 0))
               for j in range(_W)]
            + [pl.BlockSpec((2 * _G, _P), lambda i, s, o: (0, 0))]
        ),
        out_specs=pl.BlockSpec((n, 128), lambda i, s, o: (0, 0)),
    )
    out = pl.pallas_call(
        _dist_kernel,
        grid_spec=grid_spec,
        out_shape=jax.ShapeDtypeStruct((n, 128), jnp.float32),
    )(sc, order, embedding, *([protos] * _W), coords_t)

    minsq_bk = out[:, 0].reshape(B, K)
    lngs_bk = out[:, 1].reshape(B, K)
    lats_bk = out[:, 2].reshape(B, K)

    llh, pid, fprobs = pl.pallas_call(
        _epilogue_kernel,
        out_shape=[
            jax.ShapeDtypeStruct((B, 2), jnp.float32),
            jax.ShapeDtypeStruct((B, 1), jnp.int32),
            jax.ShapeDtypeStruct((B, K), jnp.float32),
        ],
    )(minsq_bk, lngs_bk, lats_bk, candidate_probs[:, :K].astype(jnp.float32),
      initial_preds)

    return llh, pid[:, 0], fprobs
